# Initial kernel scaffold; baseline (speedup 1.0000x reference)
import jax
import jax.numpy as jnp
from jax.experimental import pallas as pl


def kernel(x, edge_index, edge_attr, u, batch, W1a, b1a, W1b, b1b, W2a, b2a, W2b, b2b):
    raise NotImplementedError("write your pallas kernel here")


# speedup vs baseline: 2.3092x; 2.3092x over previous
import functools

import jax
import jax.numpy as jnp
from jax import lax
from jax.experimental import pallas as pl
from jax.experimental.pallas import tpu as pltpu
from jax.experimental.pallas import tpu_sc as plsc

N_NODES = 10000
N_EDGES = 320000
D_FEAT = 128
D_EDGE = 16
HIDDEN = 256
OUT_FEATURES = 128

NC = 2
NS = 16
LANES = 16
HH = HIDDEN // NC
CH = 80
EPT = N_EDGES // NS
NCHUNK = EPT // CH
ZROWS = N_NODES // CH
NPAD = 10240
CRED = NPAD // 128


def _xw_body(x_ref, w_ref, o_ref):
    o_ref[...] = jnp.dot(x_ref[...], w_ref[...],
                         preferred_element_type=jnp.float32)


def _compute_xw(x, w1a_x):
    blk = 1000
    return pl.pallas_call(
        _xw_body,
        grid=(N_NODES // blk,),
        in_specs=[
            pl.BlockSpec((blk, D_FEAT), lambda i: (i, 0)),
            pl.BlockSpec((D_FEAT, HIDDEN), lambda i: (0, 0)),
        ],
        out_specs=pl.BlockSpec((blk, HIDDEN), lambda i: (i, 0)),
        out_shape=jax.ShapeDtypeStruct((N_NODES, HIDDEN), jnp.float32),
    )(x, w1a_x)


def _ew_body(ea_ref, w_ref, b_ref, o_ref):
    o_ref[...] = (jnp.dot(ea_ref[...], w_ref[...],
                          preferred_element_type=jnp.float32)
                  + b_ref[...])[None]


def _compute_ew(ea, w1a_e, b1a):
    blk = 4000
    return pl.pallas_call(
        _ew_body,
        grid=(NC, N_EDGES // blk),
        in_specs=[
            pl.BlockSpec((blk, D_EDGE), lambda c, e: (e, 0)),
            pl.BlockSpec((D_EDGE, HH), lambda c, e: (0, c)),
            pl.BlockSpec((1, HH), lambda c, e: (0, c)),
        ],
        out_specs=pl.BlockSpec((1, blk, HH), lambda c, e: (c, e, 0)),
        out_shape=jax.ShapeDtypeStruct((NC, N_EDGES, HH), jnp.float32),
    )(ea, w1a_e, b1a)


def _edge_body(xwr, ew2, rowi, coli, out, cnt_out,
               rowv, idx2v, colv, gbuf, ebuf, hbuf,
               cnt_local, acc, gsem, esem):
    c = lax.axis_index("c")
    s = lax.axis_index("s")
    ones16 = jnp.ones((LANES,), jnp.float32)
    zero16 = jnp.zeros((LANES,), jnp.float32)

    def _zrow(r, carry):
        for j in range(HH // LANES):
            hbuf[r, pl.ds(LANES * j, LANES)] = zero16
        return carry
    lax.fori_loop(0, CH, _zrow, 0)

    def _zcnt(i, carry):
        cnt_local[pl.ds(LANES * i, LANES)] = zero16
        return carry
    lax.fori_loop(0, NPAD // LANES, _zcnt, 0)

    n_z = (ZROWS - s + NS - 1) // NS

    def _zacc(i, carry):
        k = s + i * NS
        pltpu.sync_copy(hbuf, acc.at[pl.ds(k * CH, CH)])
        return carry
    lax.fori_loop(0, n_z, _zacc, 0)

    plsc.subcore_barrier()

    base0 = s * EPT

    def _chunk(k, carry):
        base = base0 + k * CH
        pltpu.sync_copy(rowi.at[pl.ds(base, CH)], rowv)
        pltpu.sync_copy(coli.at[pl.ds(base, CH)], colv)
        for j in range(CH // LANES):
            sl = pl.ds(LANES * j, LANES)
            idx2v[sl] = rowv[sl] * 2 + c
            plsc.addupdate_scatter(cnt_local, [colv[sl]], ones16)
        gcp = pltpu.async_copy(xwr.at[idx2v], gbuf, gsem)
        ecp = pltpu.async_copy(ew2.at[c, pl.ds(base, CH)], ebuf, esem)
        ecp.wait()
        gcp.wait()

        def _elu(r, cc):
            for j in range(HH // LANES):
                sl = pl.ds(LANES * j, LANES)
                v = gbuf[r, sl] + ebuf[r, sl]
                hbuf[r, sl] = jnp.where(v > 0.0, v, jnp.exp(v) - 1.0)
            return cc
        lax.fori_loop(0, CH, _elu, 0)

        pltpu.sync_copy(hbuf, acc.at[colv], add=True)
        return carry
    lax.fori_loop(0, NCHUNK, _chunk, 0)

    @pl.when(c == 0)
    def _():
        pltpu.sync_copy(cnt_local, cnt_out.at[s])
    plsc.subcore_barrier()

    def _wb(i, carry):
        k = s + i * NS
        pltpu.sync_copy(acc.at[pl.ds(k * CH, CH)], hbuf)
        pltpu.sync_copy(hbuf, out.at[c, pl.ds(k * CH, CH)])
        return carry
    lax.fori_loop(0, n_z, _wb, 0)


def _edge_stage(xwr, ew2, rowi, coli):
    mesh = plsc.VectorSubcoreMesh(core_axis_name="c", subcore_axis_name="s",
                                  num_cores=NC, num_subcores=NS)
    kern = functools.partial(
        pl.kernel,
        compiler_params=pltpu.CompilerParams(needs_layout_passes=False),
        out_type=(
            jax.ShapeDtypeStruct((NC, N_NODES, HH), jnp.float32),
            jax.ShapeDtypeStruct((NS, NPAD), jnp.float32),
        ),
        mesh=mesh,
        scratch_types=[
            pltpu.VMEM((CH,), jnp.int32),
            pltpu.VMEM((CH,), jnp.int32),
            pltpu.VMEM((CH,), jnp.int32),
            pltpu.VMEM((CH, HH), jnp.float32),
            pltpu.VMEM((CH, HH), jnp.float32),
            pltpu.VMEM((CH, HH), jnp.float32),
            pltpu.VMEM((NPAD,), jnp.float32),
            pltpu.VMEM_SHARED((N_NODES, HH), jnp.float32),
            pltpu.SemaphoreType.DMA,
            pltpu.SemaphoreType.DMA,
        ],
    )(_edge_body)
    return kern(xwr, ew2, rowi, coli)


def _node_body(x_ref, s2_ref, cnt_ref, w1b0_ref, w1b1_ref, b1b_ref,
               w2ax_ref, w2at_ref, b2a_ref, w2b_ref, b2b_ref, o_ref):
    s0 = s2_ref[0]
    s1 = s2_ref[1]
    cnt = jnp.sum(cnt_ref[...], axis=1)[:, None]
    denom = jnp.maximum(cnt, 1.0)
    m0 = s0 / denom
    m1 = s1 / denom
    t = (jnp.dot(m0, w1b0_ref[...], preferred_element_type=jnp.float32)
         + jnp.dot(m1, w1b1_ref[...], preferred_element_type=jnp.float32)
         + b1b_ref[...])
    t = jnp.where(cnt > 0.0, t, 0.0)
    v = (jnp.dot(x_ref[...], w2ax_ref[...], preferred_element_type=jnp.float32)
         + jnp.dot(t, w2at_ref[...], preferred_element_type=jnp.float32)
         + b2a_ref[...])
    u = jnp.where(v > 0.0, v, jnp.exp(v) - 1.0)
    o_ref[...] = (jnp.dot(u, w2b_ref[...], preferred_element_type=jnp.float32)
                  + b2b_ref[...])


def _node_stage(x, s2, cnt, w1b, b1b, w2a, b2a, w2b, b2b):
    blk = 1000
    full = lambda shape: pl.BlockSpec(shape, lambda i: tuple(0 for _ in shape))
    return pl.pallas_call(
        _node_body,
        grid=(N_NODES // blk,),
        in_specs=[
            pl.BlockSpec((blk, D_FEAT), lambda i: (i, 0)),
            pl.BlockSpec((NC, blk, HH), lambda i: (0, i, 0)),
            pl.BlockSpec((blk, NS), lambda i: (i, 0)),
            full((HH, HIDDEN)),
            full((HH, HIDDEN)),
            full((1, HIDDEN)),
            full((D_FEAT, HIDDEN)),
            full((HIDDEN, HIDDEN)),
            full((1, HIDDEN)),
            full((HIDDEN, OUT_FEATURES)),
            full((1, OUT_FEATURES)),
        ],
        out_specs=pl.BlockSpec((blk, OUT_FEATURES), lambda i: (i, 0)),
        out_shape=jax.ShapeDtypeStruct((N_NODES, OUT_FEATURES), jnp.float32),
    )(x, s2, cnt, w1b[:HH], w1b[HH:],
      b1b.reshape(1, HIDDEN),
      w2a[:D_FEAT], w2a[D_FEAT:], b2a.reshape(1, HIDDEN),
      w2b, b2b.reshape(1, OUT_FEATURES))


@jax.jit
def kernel(x, edge_index, edge_attr, u, batch,
           W1a, b1a, W1b, b1b, W2a, b2a, W2b, b2b):
    del u, batch
    rowi = edge_index[0].astype(jnp.int32)
    coli = edge_index[1].astype(jnp.int32)

    xw = _compute_xw(x, W1a[:D_FEAT])
    xwr = xw.reshape(N_NODES * 2, HH)
    ew2 = _compute_ew(edge_attr, W1a[D_FEAT:], b1a.reshape(1, HIDDEN))
    s2, cnt_parts = _edge_stage(xwr, ew2, rowi, coli)
    return _node_stage(x, s2, cnt_parts[:, :N_NODES].T,
                       W1b, b1b, W2a, b2a, W2b, b2b)
